# baseline (speedup 1.0000x reference)
import functools

import jax
import jax.numpy as jnp
from jax import lax
from jax.experimental import pallas as pl
from jax.experimental.pallas import tpu as pltpu
from jax.experimental.pallas import tpu_sc as plsc

_NC = 2
_NS = 16
_L = 16

_CHUNK = 125
_CPAD = 128
_NBUF = 4


def _sc_segment_sum(col2d, row2d, w2d, xflat, *, n, e, d):
    dh = d // _NC
    e_per_tile = e // _NS
    nchunk = e_per_tile // _CHUNK
    napad = n + 8
    rows_per_tile = (n // _NS) // 8 * 8
    ztail = napad - rows_per_tile * _NS
    dtail = n - rows_per_tile * _NS
    zrows = 104
    nz = rows_per_tile // zrows
    hgroups = dh // _L

    mesh = plsc.VectorSubcoreMesh(core_axis_name="c", subcore_axis_name="s")

    @functools.partial(
        pl.kernel,
        out_type=jax.ShapeDtypeStruct((_NC, n, dh), jnp.float32),
        mesh=mesh,
        compiler_params=pltpu.CompilerParams(use_tc_tiling_on_sc=False),
        scratch_types=dict(
            row_v=pltpu.VMEM((nchunk, _CPAD), jnp.int32),
            cb=[pltpu.VMEM((_CPAD,), jnp.int32) for _ in range(_NBUF)],
            wb=[pltpu.VMEM((_CPAD,), jnp.float32) for _ in range(_NBUF)],
            xb=[pltpu.VMEM((_CPAD, dh), jnp.float32) for _ in range(_NBUF)],
            acc=pltpu.VMEM_SHARED((napad, dh), jnp.float32),
            csem=[pltpu.SemaphoreType.DMA for _ in range(_NBUF)],
            wsem=[pltpu.SemaphoreType.DMA for _ in range(_NBUF)],
            gsem=[pltpu.SemaphoreType.DMA for _ in range(_NBUF)],
            ssem=[pltpu.SemaphoreType.DMA for _ in range(_NBUF)],
        ),
    )
    def k(col_hbm, row_hbm, w_hbm, x_hbm, part_hbm,
          row_v, cb, wb, xb, acc, csem, wsem, gsem, ssem):
        c = lax.axis_index("c")
        s = lax.axis_index("s")
        tchunk0 = s * nchunk

        zero16 = jnp.zeros((_L,), jnp.float32)

        @pl.loop(0, zrows)
        def _(r):
            for g in range(hgroups):
                xb[0][r, pl.ds(g * _L, _L)] = zero16

        r0 = s * rows_per_tile
        for z in range(nz):
            pltpu.sync_copy(xb[0].at[pl.ds(0, zrows)],
                            acc.at[pl.ds(r0 + z * zrows, zrows)])

        @pl.when(s == _NS - 1)
        def _():
            pltpu.sync_copy(xb[0].at[pl.ds(0, ztail)],
                            acc.at[pl.ds(napad - ztail, ztail)])

        pltpu.sync_copy(row_hbm.at[pl.ds(tchunk0, nchunk)], row_v)

        plsc.subcore_barrier()

        def start_col(j, p):
            pltpu.async_copy(col_hbm.at[tchunk0 + j], cb[p], csem[p])

        def start_w(j, p):
            pltpu.async_copy(w_hbm.at[tchunk0 + j], wb[p], wsem[p])

        def wait_col(p):
            pltpu.make_async_copy(col_hbm.at[0], cb[p], csem[p]).wait()
            for g in range(_CPAD // _L):
                v = cb[p][pl.ds(g * _L, _L)]
                cb[p][pl.ds(g * _L, _L)] = v + v + c

        def wait_w(p):
            pltpu.make_async_copy(w_hbm.at[0], wb[p], wsem[p]).wait()

        def start_gather(p):
            pltpu.async_copy(x_hbm.at[cb[p]], xb[p], gsem[p])

        def wait_gather(p):
            pltpu.make_async_copy(x_hbm.at[cb[p]], xb[p], gsem[p]).wait()

        def start_scatter(j, p):
            pltpu.async_copy(xb[p], acc.at[row_v.at[j]], ssem[p], add=True)

        def wait_scatter(p):
            pltpu.make_async_copy(xb[p], acc.at[row_v.at[0]],
                                  ssem[p]).wait()

        def scale(j, p):
            for gg in range((_CHUNK + _L - 1) // _L):
                wvec = wb[p][pl.ds(gg * _L, _L)]
                for t in range(min(_L, _CHUNK - gg * _L)):
                    ee = gg * _L + t
                    w16 = jnp.full((_L,), wvec[t])
                    for g in range(hgroups):
                        xb[p][ee, pl.ds(g * _L, _L)] = (
                            xb[p][ee, pl.ds(g * _L, _L)] * w16)

        for p in range(_NBUF):
            start_col(p, p)
            start_w(p, p)
        wait_col(0)
        start_gather(0)
        wait_col(1)
        start_gather(1)

        @pl.loop(0, nchunk // _NBUF)
        def _(tt):
            jb = _NBUF * tt
            for kk in range(_NBUF):
                j = jb + kk
                p = kk
                q = (kk + 2) % _NBUF
                wait_gather(p)

                @pl.when(j + _NBUF < nchunk)
                def _():
                    start_col(j + _NBUF, p)

                @pl.when(j >= 2)
                def _():
                    wait_scatter(q)

                @pl.when(j + 2 < nchunk)
                def _():
                    wait_col(q)
                    start_gather(q)

                wait_w(p)
                scale(j, p)
                start_scatter(j, p)

                @pl.when(j + _NBUF < nchunk)
                def _():
                    start_w(j + _NBUF, p)

        wait_scatter((nchunk - 2) % _NBUF)
        wait_scatter((nchunk - 1) % _NBUF)

        plsc.subcore_barrier()

        for z in range(nz):
            rr = r0 + z * zrows
            pltpu.sync_copy(acc.at[pl.ds(rr, zrows)],
                            part_hbm.at[c, pl.ds(rr, zrows)])

        @pl.when(s == _NS - 1)
        def _():
            pltpu.sync_copy(acc.at[pl.ds(n - dtail, dtail)],
                            part_hbm.at[c, pl.ds(n - dtail, dtail)])

    return k(col2d, row2d, w2d, xflat)


def _tc_linear_body(p_ref, w0_ref, w1_ref, b_ref, o_ref):
    o_ref[...] = (
        lax.dot_general(p_ref[0], w0_ref[...],
                        dimension_numbers=(((1,), (1,)), ((), ())),
                        preferred_element_type=jnp.float32)
        + lax.dot_general(p_ref[1], w1_ref[...],
                          dimension_numbers=(((1,), (1,)), ((), ())),
                          preferred_element_type=jnp.float32)
        + b_ref[...])


def _tc_linear(parts, W, b, *, n, d_in, d_out):
    dh = d_in // _NC
    blk = 1000
    grid = (n // blk,)
    return pl.pallas_call(
        _tc_linear_body,
        grid=grid,
        in_specs=[
            pl.BlockSpec((_NC, blk, dh), lambda i: (0, i, 0)),
            pl.BlockSpec((d_out, dh), lambda i: (0, 0)),
            pl.BlockSpec((d_out, dh), lambda i: (0, 0)),
            pl.BlockSpec((1, d_out), lambda i: (0, 0)),
        ],
        out_specs=pl.BlockSpec((blk, d_out), lambda i: (i, 0)),
        out_shape=jax.ShapeDtypeStruct((n, d_out), jnp.float32),
    )(parts, W[:, :dh], W[:, dh:], b.reshape(1, d_out))


def kernel(edge_index, edge_weight, x, W, b):
    n, d_in = x.shape
    d_out = W.shape[0]
    e = edge_weight.shape[0]
    pad = ((0, 0), (0, _CPAD - _CHUNK))

    col = jnp.pad(edge_index[1].reshape(e // _CHUNK, _CHUNK), pad)
    row = jnp.pad(edge_index[0].reshape(e // _CHUNK, _CHUNK), pad,
                  constant_values=n)
    w2d = jnp.pad(edge_weight.reshape(e // _CHUNK, _CHUNK), pad)
    xflat = x.reshape(n * _NC, d_in // _NC)

    parts = _sc_segment_sum(col, row, w2d, xflat, n=n, e=e, d=d_in)
    return _tc_linear(parts, W, b, n=n, d_in=d_in, d_out=d_out)


# speedup vs baseline: 2.0417x; 2.0417x over previous
import functools

import jax
import jax.numpy as jnp
from jax import lax
from jax.experimental import pallas as pl
from jax.experimental.pallas import tpu as pltpu
from jax.experimental.pallas import tpu_sc as plsc

_NC = 2
_NS = 16
_L = 16

_CHUNK = 125
_NBUF = 4


def _sc_segment_sum(col2d, row2d, w2d, x, *, n, e, d):
    dh = d // _NC
    e_per_tile = e // _NS
    nchunk = e_per_tile // _CHUNK
    rows_per_tile = (n // _NS) // 8 * 8
    tail = n - rows_per_tile * _NS
    zrows = 104
    nz = rows_per_tile // zrows
    hgroups = dh // _L

    mesh = plsc.VectorSubcoreMesh(core_axis_name="c", subcore_axis_name="s")

    @functools.partial(
        pl.kernel,
        out_type=(jax.ShapeDtypeStruct((_NC, n, dh), jnp.float32),
                  jax.ShapeDtypeStruct((_NC, n, dh), jnp.float32)),
        mesh=mesh,
        compiler_params=pltpu.CompilerParams(use_tc_tiling_on_sc=False),
        scratch_types=dict(
            row_v=pltpu.VMEM((nchunk, _CHUNK), jnp.int32),
            cb=[pltpu.VMEM((_CHUNK,), jnp.int32) for _ in range(_NBUF)],
            wb=[pltpu.VMEM((128,), jnp.float32) for _ in range(_NBUF)],
            xb=[pltpu.VMEM((_CHUNK, dh), jnp.float32) for _ in range(_NBUF)],
            tbuf=pltpu.VMEM((zrows, d), jnp.float32),
            acc=pltpu.VMEM_SHARED((n, dh), jnp.float32),
            csem=[pltpu.SemaphoreType.DMA for _ in range(_NBUF)],
            wsem=[pltpu.SemaphoreType.DMA for _ in range(_NBUF)],
            gsem=[pltpu.SemaphoreType.DMA for _ in range(_NBUF)],
            ssem=[pltpu.SemaphoreType.DMA for _ in range(_NBUF)],
        ),
    )
    def k(col_hbm, row_hbm, w_hbm, x_hbm, part_hbm, xh_hbm,
          row_v, cb, wb, xb, tbuf, acc, csem, wsem, gsem, ssem):
        c = lax.axis_index("c")
        s = lax.axis_index("s")
        tchunk0 = s * nchunk
        xc = xh_hbm.at[c]

        zero16 = jnp.zeros((_L,), jnp.float32)

        @pl.loop(0, zrows)
        def _(r):
            for g in range(hgroups):
                xb[0][r, pl.ds(g * _L, _L)] = zero16

        r0 = s * rows_per_tile
        for z in range(nz):
            pltpu.sync_copy(xb[0].at[pl.ds(0, zrows)],
                            acc.at[pl.ds(r0 + z * zrows, zrows)])

        @pl.when(s == _NS - 1)
        def _():
            pltpu.sync_copy(xb[0].at[pl.ds(0, tail)],
                            acc.at[pl.ds(n - tail, tail)])

        pltpu.sync_copy(row_hbm.at[pl.ds(tchunk0, nchunk)], row_v)

        def build_xh(rr, cnt):
            pltpu.sync_copy(x_hbm.at[pl.ds(rr, cnt)],
                            tbuf.at[pl.ds(0, cnt)])
            pltpu.sync_copy(tbuf.at[pl.ds(0, cnt), pl.ds(c * dh, dh)],
                            xh_hbm.at[c, pl.ds(rr, cnt)])

        for z in range(nz):
            build_xh(r0 + z * zrows, zrows)

        @pl.when(s == _NS - 1)
        def _():
            build_xh(n - tail, tail)

        plsc.subcore_barrier()

        def start_col(j, p):
            pltpu.async_copy(col_hbm.at[tchunk0 + j], cb[p], csem[p])

        def start_w(j, p):
            pltpu.async_copy(w_hbm.at[tchunk0 + j], wb[p], wsem[p])

        def wait_col(p):
            pltpu.make_async_copy(col_hbm.at[0], cb[p], csem[p]).wait()

        def wait_w(p):
            pltpu.make_async_copy(w_hbm.at[0], wb[p], wsem[p]).wait()

        def start_gather(p):
            pltpu.async_copy(xc.at[cb[p]], xb[p], gsem[p])

        def wait_gather(p):
            pltpu.make_async_copy(xc.at[cb[p]], xb[p], gsem[p]).wait()

        def start_scatter(j, p):
            pltpu.async_copy(xb[p], acc.at[row_v.at[j]], ssem[p], add=True)

        def wait_scatter(p):
            pltpu.make_async_copy(xb[p], acc.at[row_v.at[0]],
                                  ssem[p]).wait()

        def scale(j, p):
            for gg in range((_CHUNK + _L - 1) // _L):
                wvec = wb[p][pl.ds(gg * _L, _L)]
                for t in range(min(_L, _CHUNK - gg * _L)):
                    ee = gg * _L + t
                    w16 = jnp.full((_L,), wvec[t])
                    for g in range(hgroups):
                        xb[p][ee, pl.ds(g * _L, _L)] = (
                            xb[p][ee, pl.ds(g * _L, _L)] * w16)

        for p in range(_NBUF):
            start_col(p, p)
            start_w(p, p)
        wait_col(0)
        start_gather(0)
        wait_col(1)
        start_gather(1)

        @pl.loop(0, nchunk // _NBUF)
        def _(tt):
            jb = _NBUF * tt
            for kk in range(_NBUF):
                j = jb + kk
                p = kk
                q = (kk + 2) % _NBUF
                wait_gather(p)

                @pl.when(j + _NBUF < nchunk)
                def _():
                    start_col(j + _NBUF, p)

                @pl.when(j >= 2)
                def _():
                    wait_scatter(q)

                @pl.when(j + 2 < nchunk)
                def _():
                    wait_col(q)
                    start_gather(q)

                wait_w(p)
                scale(j, p)
                start_scatter(j, p)

                @pl.when(j + _NBUF < nchunk)
                def _():
                    start_w(j + _NBUF, p)

        wait_scatter((nchunk - 2) % _NBUF)
        wait_scatter((nchunk - 1) % _NBUF)

        plsc.subcore_barrier()

        for z in range(nz):
            rr = r0 + z * zrows
            pltpu.sync_copy(acc.at[pl.ds(rr, zrows)],
                            part_hbm.at[c, pl.ds(rr, zrows)])

        @pl.when(s == _NS - 1)
        def _():
            pltpu.sync_copy(acc.at[pl.ds(n - tail, tail)],
                            part_hbm.at[c, pl.ds(n - tail, tail)])

    return k(col2d, row2d, w2d, x)[0]


def _tc_linear_body(p_ref, w0_ref, w1_ref, b_ref, o_ref):
    o_ref[...] = (
        lax.dot_general(p_ref[0], w0_ref[...],
                        dimension_numbers=(((1,), (1,)), ((), ())),
                        preferred_element_type=jnp.float32)
        + lax.dot_general(p_ref[1], w1_ref[...],
                          dimension_numbers=(((1,), (1,)), ((), ())),
                          preferred_element_type=jnp.float32)
        + b_ref[...])


def _tc_linear(parts, W, b, *, n, d_in, d_out):
    dh = d_in // _NC
    blk = 1000
    grid = (n // blk,)
    return pl.pallas_call(
        _tc_linear_body,
        grid=grid,
        in_specs=[
            pl.BlockSpec((_NC, blk, dh), lambda i: (0, i, 0)),
            pl.BlockSpec((d_out, dh), lambda i: (0, 0)),
            pl.BlockSpec((d_out, dh), lambda i: (0, 0)),
            pl.BlockSpec((1, d_out), lambda i: (0, 0)),
        ],
        out_specs=pl.BlockSpec((blk, d_out), lambda i: (i, 0)),
        out_shape=jax.ShapeDtypeStruct((n, d_out), jnp.float32),
    )(parts, W[:, :dh], W[:, dh:], b.reshape(1, d_out))


def kernel(edge_index, edge_weight, x, W, b):
    n, d_in = x.shape
    d_out = W.shape[0]
    e = edge_weight.shape[0]

    row = edge_index[0].reshape(e // _CHUNK, _CHUNK)
    col = edge_index[1].reshape(e // _CHUNK, _CHUNK)
    w2d = jnp.pad(edge_weight.reshape(e // _CHUNK, _CHUNK),
                  ((0, 0), (0, 128 - _CHUNK)))

    parts = _sc_segment_sum(col, row, w2d, x, n=n, e=e, d=d_in)
    return _tc_linear(parts, W, b, n=n, d_in=d_in, d_out=d_out)
